# baseline (speedup 1.0000x reference)
import functools

import jax
import jax.numpy as jnp
from jax import lax
from jax.experimental import pallas as pl
from jax.experimental.pallas import tpu as pltpu
from jax.experimental.pallas import tpu_sc as plsc

DIM = 128
MAT = 32
B = 4096
NNEG = 50
MARGIN = 2.0
ALPHA = 0.01
BETA = 0.01
C_MIN = 1.0
C_MAX = 100.0
LAM = 0.75

NC = 2
NS = 16
NW = NC * NS
BPW = B // NW


def _rsqrt16(x):
    i = lax.bitcast_convert_type(x, jnp.int32)
    i = 0x5F3759DF - lax.shift_right_arithmetic(i, 1)
    y = lax.bitcast_convert_type(i, jnp.float32)
    xh = 0.5 * x
    for _ in range(3):
        y = y * (1.5 - xh * y * y)
    return y


def _scale16(nrm2):
    n = nrm2 * _rsqrt16(nrm2)
    return jnp.where(n > 1.0, 1.0 / (n + 1e-7), 1.0)


NBUF = 4


def _sc_loss0_body(uid_hbm, pid_hbm, neg_hbm, uemb_hbm, iemb_hbm, out_hbm,
                   uid_v, pid_v, neg_v, ue_v, pe_v, rows0, rows1, rows2,
                   rows3, hout_v, sem0, sem1, sem2, sem3, semg):
    wid = lax.axis_index("s") * NC + lax.axis_index("c")
    base = wid * BPW

    pltpu.sync_copy(uid_hbm.at[pl.ds(base, BPW)], uid_v)
    pltpu.sync_copy(pid_hbm.at[pl.ds(base, BPW)], pid_v)
    pltpu.sync_copy(neg_hbm.at[pl.ds(wid * (BPW // 2), BPW // 2)], neg_v)
    pltpu.async_copy(uemb_hbm.at[uid_v], ue_v, semg).wait()
    pltpu.async_copy(iemb_hbm.at[pid_v], pe_v, semg).wait()

    iota = lax.iota(jnp.int32, 16)
    perms = [iota ^ sh for sh in (8, 4, 2, 1)]
    blk = lax.shift_right_arithmetic(iota, 2)
    blk0, blk1, blk2 = blk == 0, blk == 1, blk == 2

    def tsum(a):
        for p in perms:
            a = a + a.at[p].get(mode="promise_in_bounds")
        return a

    zf = jnp.zeros((16,), jnp.float32)

    bufs = [rows0, rows1, rows2, rows3]
    sems = [sem0, sem1, sem2, sem3]

    for b0 in range(NBUF - 1):
        pltpu.async_copy(iemb_hbm.at[neg_v.at[b0]], bufs[b0], sems[b0])

    def half_b(bb, roff, cur, hvec):
        us = [ue_v[bb, pl.ds(16 * k, 16)] for k in range(DIM // 16)]
        ps = [pe_v[bb, pl.ds(16 * k, 16)] for k in range(DIM // 16)]
        accU, accP, accD = zf, zf, zf
        for k in range(DIM // 16):
            accU = accU + us[k] * us[k]
            accP = accP + ps[k] * ps[k]
            accD = accD + us[k] * ps[k]
        U, P, D = tsum(accU), tsum(accP), tsum(accD)
        su, sp = _scale16(U), _scale16(P)
        nu = su * su * U
        dp = nu - 2.0 * su * sp * D + sp * sp * P

        def reduce2(a):
            a = a + a.at[perms[0]].get(mode="promise_in_bounds")
            return a + a.at[perms[1]].get(mode="promise_in_bounds")

        def finish2(a):
            a = a + a.at[perms[2]].get(mode="promise_in_bounds")
            return a + a.at[perms[3]].get(mode="promise_in_bounds")

        def quad(rows, mmin):
            ds_, ns_ = [], []
            for r0 in rows:
                r = r0 + roff
                accd = accn = zf
                for k in range(DIM // 16):
                    v = cur[r, pl.ds(16 * k, 16)]
                    accd = accd + v * us[k]
                    accn = accn + v * v
                ds_.append(reduce2(accd))
                ns_.append(reduce2(accn))
            dsv = jnp.where(blk0, ds_[0], jnp.where(blk1, ds_[1],
                            jnp.where(blk2, ds_[2], ds_[3])))
            nsv = jnp.where(blk0, ns_[0], jnp.where(blk1, ns_[1],
                            jnp.where(blk2, ns_[2], ns_[3])))
            dsv, nsv = finish2(dsv), finish2(nsv)
            s = _scale16(nsv)
            d = nu - 2.0 * su * s * dsv + s * s * nsv
            return jnp.minimum(mmin, d)

        @plsc.parallel_loop(0, NNEG // 4, unroll=2,
                            carry=jnp.full((16,), 3.4e38, jnp.float32))
        def rloop(j, mmin):
            r = 4 * j
            return quad([r, r + 1, r + 2, r + 3], mmin)

        mmin = quad([NNEG - 2, NNEG - 1, NNEG - 1, NNEG - 1], rloop)
        mmin = jnp.minimum(mmin, mmin.at[perms[0]].get(mode="promise_in_bounds"))
        mmin = jnp.minimum(mmin, mmin.at[perms[1]].get(mode="promise_in_bounds"))
        return hvec + jnp.maximum(MARGIN + dp - mmin, 0.0)

    NP2 = BPW // 2

    def one_pair(pp, hvec, cur, cur_sem, nxt, nxt_sem):
        @pl.when(pp + NBUF - 1 < NP2)
        def _():
            pltpu.async_copy(iemb_hbm.at[neg_v.at[pp + NBUF - 1]], nxt, nxt_sem)
        pltpu.make_async_copy(iemb_hbm.at[neg_v.at[pp]], cur, cur_sem).wait()
        hvec = half_b(2 * pp, 0, cur, hvec)
        hvec = half_b(2 * pp + 1, NNEG, cur, hvec)
        return hvec

    def ring_body(i, hvec):
        for phase in range(NBUF):
            pp = NBUF * i + phase
            hvec = one_pair(pp, hvec, bufs[phase], sems[phase],
                            bufs[(phase + NBUF - 1) % NBUF],
                            sems[(phase + NBUF - 1) % NBUF])
        return hvec

    hvec = lax.fori_loop(0, NP2 // NBUF, ring_body, zf)
    hout_v[...] = hvec
    pltpu.sync_copy(hout_v, out_hbm.at[wid])


@jax.jit
def _sc_loss0(user_ids, pos_ids, neg_ids, user_emb, item_emb):
    mesh = plsc.VectorSubcoreMesh(core_axis_name="c", subcore_axis_name="s",
                                  num_cores=NC, num_subcores=NS)
    f = pl.kernel(
        _sc_loss0_body,
        out_type=jax.ShapeDtypeStruct((NW, 16), jnp.float32),
        mesh=mesh,
        scratch_types=[
            pltpu.VMEM((BPW,), jnp.int32),
            pltpu.VMEM((BPW,), jnp.int32),
            pltpu.VMEM((BPW // 2, 2 * NNEG), jnp.int32),
            pltpu.VMEM((BPW, DIM), jnp.float32),
            pltpu.VMEM((BPW, DIM), jnp.float32),
            pltpu.VMEM((2 * NNEG, DIM), jnp.float32),
            pltpu.VMEM((2 * NNEG, DIM), jnp.float32),
            pltpu.VMEM((2 * NNEG, DIM), jnp.float32),
            pltpu.VMEM((2 * NNEG, DIM), jnp.float32),
            pltpu.VMEM((16,), jnp.float32),
            pltpu.SemaphoreType.DMA,
            pltpu.SemaphoreType.DMA,
            pltpu.SemaphoreType.DMA,
            pltpu.SemaphoreType.DMA,
            pltpu.SemaphoreType.DMA,
        ],
    )
    return f(user_ids, pos_ids, neg_ids.reshape(B // 2, 2 * NNEG),
             user_emb, item_emb)


def _tc_loss12_body(uid_ref, pid_ref, ue32_ref, ie32_ref, cu32_ref, ci32_ref,
                    cub_ref, cib_ref, inter_ref, out_ref):
    def renorm(rows):
        n = jnp.sqrt(jnp.sum(rows * rows, axis=1, keepdims=True))
        return rows * jnp.where(n > 1.0, 1.0 / (n + 1e-7), 1.0)

    def presence(ids_col):
        v = lax.broadcasted_iota(jnp.int32, (B, MAT), 1)
        return jnp.max((ids_col == v).astype(jnp.float32), axis=0,
                       keepdims=True)

    def glove(p_row, emb32, co32, bias_row, binmat, coef):
        n = jnp.sum(p_row)
        C = jnp.dot(binmat, binmat.T, preferred_element_type=jnp.float32)
        co = jnp.where(C > 0.0, C, 1.0)
        eye = (lax.broadcasted_iota(jnp.int32, (MAT, MAT), 0) ==
               lax.broadcasted_iota(jnp.int32, (MAT, MAT), 1)).astype(jnp.float32)
        co = co * (1.0 - eye) + eye
        w = jnp.exp(LAM * jnp.log(co / C_MAX))
        w = jnp.where(co <= C_MIN, 0.0, w)
        w = jnp.where(co >= C_MAX, 1.0, w)
        cx = renorm(co32)
        up = jnp.dot(cx, cx.T, preferred_element_type=jnp.float32)
        E = up + bias_row.reshape(MAT, 1) - jnp.log(co)
        S1 = jnp.sum(p_row * bias_row)
        S2 = jnp.sum(p_row * bias_row * bias_row)
        p_col = p_row.reshape(MAT, 1)
        core = jnp.sum(p_col * p_row * w *
                       (n * E * E + 2.0 * S1 * E + S2)) / (n * n * n)
        ex = renorm(emb32)
        cons = coef * jnp.sum(p_col * (ex - cx) ** 2) / (n * DIM)
        return core + cons

    binmat = (inter_ref[...] != 0.0).astype(jnp.float32)
    pu = presence(uid_ref[...])
    pi = presence(pid_ref[...])
    l1 = glove(pu, ue32_ref[...], cu32_ref[...], cub_ref[...], binmat, ALPHA)
    l2 = glove(pi, ie32_ref[...], ci32_ref[...], cib_ref[...], binmat.T, BETA)
    out_ref[...] = jnp.broadcast_to(l1 + l2, (1, 1))


@jax.jit
def _tc_loss12(uid_col, pid_col, ue32, ie32, cu32, ci32, cub_row, cib_row,
               interactions):
    return pl.pallas_call(
        _tc_loss12_body,
        out_shape=jax.ShapeDtypeStruct((1, 1), jnp.float32),
    )(uid_col, pid_col, ue32, ie32, cu32, ci32, cub_row, cib_row, interactions)


def kernel(user_ids, pos_ids, neg_ids, user_emb, item_emb, co_user_emb,
           co_item_emb, co_user_bias, co_item_bias, interactions):
    loss0_parts = _sc_loss0(user_ids, pos_ids, neg_ids, user_emb, item_emb)
    loss12 = _tc_loss12(
        user_ids.reshape(B, 1), pos_ids.reshape(B, 1),
        user_emb[:MAT], item_emb[:MAT], co_user_emb[:MAT], co_item_emb[:MAT],
        co_user_bias[:MAT].reshape(1, MAT), co_item_bias[:MAT].reshape(1, MAT),
        interactions)
    return jnp.sum(loss0_parts) / (16.0 * B) + loss12[0, 0]


# speedup vs baseline: 1.0283x; 1.0283x over previous
import functools

import jax
import jax.numpy as jnp
from jax import lax
from jax.experimental import pallas as pl
from jax.experimental.pallas import tpu as pltpu
from jax.experimental.pallas import tpu_sc as plsc

DIM = 128
MAT = 32
B = 4096
NNEG = 50
MARGIN = 2.0
ALPHA = 0.01
BETA = 0.01
C_MIN = 1.0
C_MAX = 100.0
LAM = 0.75

NC = 2
NS = 16
NW = NC * NS
BPW = B // NW


def _rsqrt16(x):
    i = lax.bitcast_convert_type(x, jnp.int32)
    i = 0x5F3759DF - lax.shift_right_arithmetic(i, 1)
    y = lax.bitcast_convert_type(i, jnp.float32)
    xh = 0.5 * x
    for _ in range(3):
        y = y * (1.5 - xh * y * y)
    return y


def _scale16(nrm2):
    n = nrm2 * _rsqrt16(nrm2)
    return jnp.where(n > 1.0, 1.0 / (n + 1e-7), 1.0)


NBUF = 6


def _sc_loss0_body(uid_hbm, pid_hbm, neg_hbm, uemb_hbm, iemb_hbm, out_hbm,
                   uid_v, pid_v, neg_v, ue_v, pe_v, rows0, rows1, rows2,
                   rows3, rows4, rows5, hout_v,
                   sem0, sem1, sem2, sem3, sem4, sem5, semg):
    wid = lax.axis_index("s") * NC + lax.axis_index("c")
    base = wid * BPW

    pltpu.sync_copy(uid_hbm.at[pl.ds(base, BPW)], uid_v)
    pltpu.sync_copy(pid_hbm.at[pl.ds(base, BPW)], pid_v)
    pltpu.sync_copy(neg_hbm.at[pl.ds(base, BPW)], neg_v)
    pltpu.async_copy(uemb_hbm.at[uid_v], ue_v, semg).wait()
    pltpu.async_copy(iemb_hbm.at[pid_v], pe_v, semg).wait()

    iota = lax.iota(jnp.int32, 16)
    perms = [iota ^ sh for sh in (8, 4, 2, 1)]
    blk = lax.shift_right_arithmetic(iota, 2)
    blk0, blk1, blk2 = blk == 0, blk == 1, blk == 2

    def tsum(a):
        for p in perms:
            a = a + a.at[p].get(mode="promise_in_bounds")
        return a

    zf = jnp.zeros((16,), jnp.float32)

    bufs = [rows0, rows1, rows2, rows3, rows4, rows5]
    sems = [sem0, sem1, sem2, sem3, sem4, sem5]

    for b0 in range(NBUF - 1):
        pltpu.async_copy(iemb_hbm.at[neg_v.at[b0]], bufs[b0], sems[b0])

    def one_b(bb, hvec, cur, cur_sem, nxt, nxt_sem):
        @pl.when(bb + NBUF - 1 < BPW)
        def _():
            pltpu.async_copy(iemb_hbm.at[neg_v.at[bb + NBUF - 1]], nxt, nxt_sem)

        us = [ue_v[bb, pl.ds(16 * k, 16)] for k in range(DIM // 16)]
        ps = [pe_v[bb, pl.ds(16 * k, 16)] for k in range(DIM // 16)]
        accU, accP, accD = zf, zf, zf
        for k in range(DIM // 16):
            accU = accU + us[k] * us[k]
            accP = accP + ps[k] * ps[k]
            accD = accD + us[k] * ps[k]
        U, P, D = tsum(accU), tsum(accP), tsum(accD)
        su, sp = _scale16(U), _scale16(P)
        nu = su * su * U
        dp = nu - 2.0 * su * sp * D + sp * sp * P

        pltpu.make_async_copy(iemb_hbm.at[neg_v.at[bb]], cur, cur_sem).wait()

        def reduce2(a):
            a = a + a.at[perms[0]].get(mode="promise_in_bounds")
            return a + a.at[perms[1]].get(mode="promise_in_bounds")

        def finish2(a):
            a = a + a.at[perms[2]].get(mode="promise_in_bounds")
            return a + a.at[perms[3]].get(mode="promise_in_bounds")

        def quad(rows, mmin):
            ds_, ns_ = [], []
            for r in rows:
                accd = accn = zf
                for k in range(DIM // 16):
                    v = cur[r, pl.ds(16 * k, 16)]
                    accd = accd + v * us[k]
                    accn = accn + v * v
                ds_.append(reduce2(accd))
                ns_.append(reduce2(accn))
            dsv = jnp.where(blk0, ds_[0], jnp.where(blk1, ds_[1],
                            jnp.where(blk2, ds_[2], ds_[3])))
            nsv = jnp.where(blk0, ns_[0], jnp.where(blk1, ns_[1],
                            jnp.where(blk2, ns_[2], ns_[3])))
            dsv, nsv = finish2(dsv), finish2(nsv)
            s = _scale16(nsv)
            d = nu - 2.0 * su * s * dsv + s * s * nsv
            return jnp.minimum(mmin, d)

        @plsc.parallel_loop(0, NNEG // 4, unroll=2,
                            carry=jnp.full((16,), 3.4e38, jnp.float32))
        def rloop(j, mmin):
            r = 4 * j
            return quad([r, r + 1, r + 2, r + 3], mmin)

        mmin = quad([NNEG - 2, NNEG - 1, NNEG - 1, NNEG - 1], rloop)
        mmin = jnp.minimum(mmin, mmin.at[perms[0]].get(mode="promise_in_bounds"))
        mmin = jnp.minimum(mmin, mmin.at[perms[1]].get(mode="promise_in_bounds"))
        return hvec + jnp.maximum(MARGIN + dp - mmin, 0.0)

    def ring_body(i, hvec):
        for phase in range(NBUF):
            bb = NBUF * i + phase
            hvec = one_b(bb, hvec, bufs[phase], sems[phase],
                         bufs[(phase + NBUF - 1) % NBUF],
                         sems[(phase + NBUF - 1) % NBUF])
        return hvec

    hvec = lax.fori_loop(0, BPW // NBUF, ring_body, zf)
    for bb in range(NBUF * (BPW // NBUF), BPW):
        ph = bb % NBUF
        hvec = one_b(bb, hvec, bufs[ph], sems[ph],
                     bufs[(ph + NBUF - 1) % NBUF], sems[(ph + NBUF - 1) % NBUF])
    hout_v[...] = hvec
    pltpu.sync_copy(hout_v, out_hbm.at[wid])


@jax.jit
def _sc_loss0(user_ids, pos_ids, neg_ids, user_emb, item_emb):
    mesh = plsc.VectorSubcoreMesh(core_axis_name="c", subcore_axis_name="s",
                                  num_cores=NC, num_subcores=NS)
    f = pl.kernel(
        _sc_loss0_body,
        out_type=jax.ShapeDtypeStruct((NW, 16), jnp.float32),
        mesh=mesh,
        scratch_types=[
            pltpu.VMEM((BPW,), jnp.int32),
            pltpu.VMEM((BPW,), jnp.int32),
            pltpu.VMEM((BPW, NNEG), jnp.int32),
            pltpu.VMEM((BPW, DIM), jnp.float32),
            pltpu.VMEM((BPW, DIM), jnp.float32),
            pltpu.VMEM((NNEG, DIM), jnp.float32),
            pltpu.VMEM((NNEG, DIM), jnp.float32),
            pltpu.VMEM((NNEG, DIM), jnp.float32),
            pltpu.VMEM((NNEG, DIM), jnp.float32),
            pltpu.VMEM((NNEG, DIM), jnp.float32),
            pltpu.VMEM((NNEG, DIM), jnp.float32),
            pltpu.VMEM((16,), jnp.float32),
            pltpu.SemaphoreType.DMA,
            pltpu.SemaphoreType.DMA,
            pltpu.SemaphoreType.DMA,
            pltpu.SemaphoreType.DMA,
            pltpu.SemaphoreType.DMA,
            pltpu.SemaphoreType.DMA,
            pltpu.SemaphoreType.DMA,
        ],
    )
    return f(user_ids, pos_ids, neg_ids, user_emb, item_emb)


def _tc_loss12_body(uid_ref, pid_ref, ue32_ref, ie32_ref, cu32_ref, ci32_ref,
                    cub_ref, cib_ref, inter_ref, out_ref):
    def renorm(rows):
        n = jnp.sqrt(jnp.sum(rows * rows, axis=1, keepdims=True))
        return rows * jnp.where(n > 1.0, 1.0 / (n + 1e-7), 1.0)

    def presence(ids_col):
        v = lax.broadcasted_iota(jnp.int32, (B, MAT), 1)
        return jnp.max((ids_col == v).astype(jnp.float32), axis=0,
                       keepdims=True)

    def glove(p_row, emb32, co32, bias_row, binmat, coef):
        n = jnp.sum(p_row)
        C = jnp.dot(binmat, binmat.T, preferred_element_type=jnp.float32)
        co = jnp.where(C > 0.0, C, 1.0)
        eye = (lax.broadcasted_iota(jnp.int32, (MAT, MAT), 0) ==
               lax.broadcasted_iota(jnp.int32, (MAT, MAT), 1)).astype(jnp.float32)
        co = co * (1.0 - eye) + eye
        w = jnp.exp(LAM * jnp.log(co / C_MAX))
        w = jnp.where(co <= C_MIN, 0.0, w)
        w = jnp.where(co >= C_MAX, 1.0, w)
        cx = renorm(co32)
        up = jnp.dot(cx, cx.T, preferred_element_type=jnp.float32)
        E = up + bias_row.reshape(MAT, 1) - jnp.log(co)
        S1 = jnp.sum(p_row * bias_row)
        S2 = jnp.sum(p_row * bias_row * bias_row)
        p_col = p_row.reshape(MAT, 1)
        core = jnp.sum(p_col * p_row * w *
                       (n * E * E + 2.0 * S1 * E + S2)) / (n * n * n)
        ex = renorm(emb32)
        cons = coef * jnp.sum(p_col * (ex - cx) ** 2) / (n * DIM)
        return core + cons

    binmat = (inter_ref[...] != 0.0).astype(jnp.float32)
    pu = presence(uid_ref[...])
    pi = presence(pid_ref[...])
    l1 = glove(pu, ue32_ref[...], cu32_ref[...], cub_ref[...], binmat, ALPHA)
    l2 = glove(pi, ie32_ref[...], ci32_ref[...], cib_ref[...], binmat.T, BETA)
    out_ref[...] = jnp.broadcast_to(l1 + l2, (1, 1))


@jax.jit
def _tc_loss12(uid_col, pid_col, ue32, ie32, cu32, ci32, cub_row, cib_row,
               interactions):
    return pl.pallas_call(
        _tc_loss12_body,
        out_shape=jax.ShapeDtypeStruct((1, 1), jnp.float32),
    )(uid_col, pid_col, ue32, ie32, cu32, ci32, cub_row, cib_row, interactions)


def kernel(user_ids, pos_ids, neg_ids, user_emb, item_emb, co_user_emb,
           co_item_emb, co_user_bias, co_item_bias, interactions):
    loss0_parts = _sc_loss0(user_ids, pos_ids, neg_ids, user_emb, item_emb)
    loss12 = _tc_loss12(
        user_ids.reshape(B, 1), pos_ids.reshape(B, 1),
        user_emb[:MAT], item_emb[:MAT], co_user_emb[:MAT], co_item_emb[:MAT],
        co_user_bias[:MAT].reshape(1, MAT), co_item_bias[:MAT].reshape(1, MAT),
        interactions)
    return jnp.sum(loss0_parts) / (16.0 * B) + loss12[0, 0]


# speedup vs baseline: 1.2570x; 1.2224x over previous
import jax
import jax.numpy as jnp
from jax import lax
from jax.experimental import pallas as pl
from jax.experimental.pallas import tpu as pltpu
from jax.experimental.pallas import tpu_sc as plsc

DIM = 128
MAT = 32
B = 4096
NNEG = 50
MARGIN = 2.0
ALPHA = 0.01
BETA = 0.01
C_MIN = 1.0
C_MAX = 100.0
LAM = 0.75

NC = 2
NS = 16
NW = NC * NS
BPW = B // NW


def _rsqrt16(x):
    i = lax.bitcast_convert_type(x, jnp.int32)
    i = 0x5F3759DF - lax.shift_right_arithmetic(i, 1)
    y = lax.bitcast_convert_type(i, jnp.float32)
    xh = 0.5 * x
    for _ in range(3):
        y = y * (1.5 - xh * y * y)
    return y


def _scale16(nrm2):
    n = nrm2 * _rsqrt16(nrm2)
    return jnp.where(n > 1.0, 1.0 / (n + 1e-7), 1.0)


NBUF = 4


def _sc_loss0_body(uid_hbm, pid_hbm, neg_hbm, uemb_hbm, iemb_hbm, out_hbm,
                   uid_v, pid_v, neg_v, ue_v, pe_v, rows0, rows1, rows2,
                   rows3, hout_v, sem0, sem1, sem2, sem3, semg):
    wid = lax.axis_index("s") * NC + lax.axis_index("c")
    base = wid * BPW

    pltpu.sync_copy(uid_hbm.at[pl.ds(base, BPW)], uid_v)
    pltpu.sync_copy(pid_hbm.at[pl.ds(base, BPW)], pid_v)
    pltpu.sync_copy(neg_hbm.at[pl.ds(base, BPW)], neg_v)
    pltpu.async_copy(uemb_hbm.at[uid_v], ue_v, semg).wait()
    pltpu.async_copy(iemb_hbm.at[pid_v], pe_v, semg).wait()

    iota = lax.iota(jnp.int32, 16)
    perms = [iota ^ sh for sh in (8, 4, 2, 1)]
    blk = lax.shift_right_arithmetic(iota, 2)
    blk0, blk1, blk2 = blk == 0, blk == 1, blk == 2

    def tsum(a):
        for p in perms:
            a = a + a.at[p].get(mode="promise_in_bounds")
        return a

    zf = jnp.zeros((16,), jnp.float32)

    bufs = [rows0, rows1, rows2, rows3]
    sems = [sem0, sem1, sem2, sem3]

    for b0 in range(NBUF - 1):
        pltpu.async_copy(iemb_hbm.at[neg_v.at[b0]], bufs[b0], sems[b0])

    def one_b(bb, hvec, cur, cur_sem, nxt, nxt_sem):
        @pl.when(bb + NBUF - 1 < BPW)
        def _():
            pltpu.async_copy(iemb_hbm.at[neg_v.at[bb + NBUF - 1]], nxt, nxt_sem)

        us = [ue_v[bb, pl.ds(16 * k, 16)] for k in range(DIM // 16)]
        ps = [pe_v[bb, pl.ds(16 * k, 16)] for k in range(DIM // 16)]
        accU, accP, accD = zf, zf, zf
        for k in range(DIM // 16):
            accU = accU + us[k] * us[k]
            accP = accP + ps[k] * ps[k]
            accD = accD + us[k] * ps[k]
        U, P, D = tsum(accU), tsum(accP), tsum(accD)
        su, sp = _scale16(U), _scale16(P)
        nu = su * su * U
        dp = nu - 2.0 * su * sp * D + sp * sp * P

        pltpu.make_async_copy(iemb_hbm.at[neg_v.at[bb]], cur, cur_sem).wait()

        def reduce2(a):
            a = a + a.at[perms[0]].get(mode="promise_in_bounds")
            return a + a.at[perms[1]].get(mode="promise_in_bounds")

        def finish2(a):
            a = a + a.at[perms[2]].get(mode="promise_in_bounds")
            return a + a.at[perms[3]].get(mode="promise_in_bounds")

        def quad(rows, mmin):
            ds_, ns_ = [], []
            for r in rows:
                accd = accn = zf
                for k in range(DIM // 16):
                    v = cur[r, pl.ds(16 * k, 16)]
                    accd = accd + v * us[k]
                    accn = accn + v * v
                ds_.append(reduce2(accd))
                ns_.append(reduce2(accn))
            dsv = jnp.where(blk0, ds_[0], jnp.where(blk1, ds_[1],
                            jnp.where(blk2, ds_[2], ds_[3])))
            nsv = jnp.where(blk0, ns_[0], jnp.where(blk1, ns_[1],
                            jnp.where(blk2, ns_[2], ns_[3])))
            dsv, nsv = finish2(dsv), finish2(nsv)
            s = _scale16(nsv)
            d = nu - 2.0 * su * s * dsv + s * s * nsv
            return jnp.minimum(mmin, d)

        @plsc.parallel_loop(0, NNEG // 4, unroll=2,
                            carry=jnp.full((16,), 3.4e38, jnp.float32))
        def rloop(j, mmin):
            r = 4 * j
            return quad([r, r + 1, r + 2, r + 3], mmin)

        mmin = quad([NNEG - 2, NNEG - 1, NNEG - 1, NNEG - 1], rloop)
        mmin = jnp.minimum(mmin, mmin.at[perms[0]].get(mode="promise_in_bounds"))
        mmin = jnp.minimum(mmin, mmin.at[perms[1]].get(mode="promise_in_bounds"))
        return hvec + jnp.maximum(MARGIN + dp - mmin, 0.0)

    def ring_body(i, hvec):
        for phase in range(NBUF):
            bb = NBUF * i + phase
            hvec = one_b(bb, hvec, bufs[phase], sems[phase],
                         bufs[(phase + NBUF - 1) % NBUF],
                         sems[(phase + NBUF - 1) % NBUF])
        return hvec

    hvec = lax.fori_loop(0, BPW // NBUF, ring_body, zf)
    hout_v[...] = hvec
    pltpu.sync_copy(hout_v, out_hbm.at[wid])


@jax.jit
def _sc_loss0(user_ids, pos_ids, neg_ids, user_emb, item_emb):
    mesh = plsc.VectorSubcoreMesh(core_axis_name="c", subcore_axis_name="s",
                                  num_cores=NC, num_subcores=NS)
    f = pl.kernel(
        _sc_loss0_body,
        out_type=jax.ShapeDtypeStruct((NW, 16), jnp.float32),
        mesh=mesh,
        scratch_types=[
            pltpu.VMEM((BPW,), jnp.int32),
            pltpu.VMEM((BPW,), jnp.int32),
            pltpu.VMEM((BPW, NNEG), jnp.int32),
            pltpu.VMEM((BPW, DIM), jnp.float32),
            pltpu.VMEM((BPW, DIM), jnp.float32),
            pltpu.VMEM((NNEG, DIM), jnp.float32),
            pltpu.VMEM((NNEG, DIM), jnp.float32),
            pltpu.VMEM((NNEG, DIM), jnp.float32),
            pltpu.VMEM((NNEG, DIM), jnp.float32),
            pltpu.VMEM((16,), jnp.float32),
            pltpu.SemaphoreType.DMA,
            pltpu.SemaphoreType.DMA,
            pltpu.SemaphoreType.DMA,
            pltpu.SemaphoreType.DMA,
            pltpu.SemaphoreType.DMA,
        ],
    )
    return f(user_ids, pos_ids, neg_ids, user_emb, item_emb)


def _tc_loss12_body(uid_ref, pid_ref, ue32_ref, ie32_ref, cu32_ref, ci32_ref,
                    cub_ref, cib_ref, inter_ref, out_ref):
    def renorm(rows):
        n = jnp.sqrt(jnp.sum(rows * rows, axis=1, keepdims=True))
        return rows * jnp.where(n > 1.0, 1.0 / (n + 1e-7), 1.0)

    def presence(ids_col):
        v = lax.broadcasted_iota(jnp.int32, (B, MAT), 1)
        return jnp.max((ids_col == v).astype(jnp.float32), axis=0,
                       keepdims=True)

    def glove(p_row, emb32, co32, bias_row, binmat, coef):
        n = jnp.sum(p_row)
        C = jnp.dot(binmat, binmat.T, preferred_element_type=jnp.float32)
        co = jnp.where(C > 0.0, C, 1.0)
        eye = (lax.broadcasted_iota(jnp.int32, (MAT, MAT), 0) ==
               lax.broadcasted_iota(jnp.int32, (MAT, MAT), 1)).astype(jnp.float32)
        co = co * (1.0 - eye) + eye
        w = jnp.exp(LAM * jnp.log(co / C_MAX))
        w = jnp.where(co <= C_MIN, 0.0, w)
        w = jnp.where(co >= C_MAX, 1.0, w)
        cx = renorm(co32)
        up = jnp.dot(cx, cx.T, preferred_element_type=jnp.float32)
        E = up + bias_row.reshape(MAT, 1) - jnp.log(co)
        S1 = jnp.sum(p_row * bias_row)
        S2 = jnp.sum(p_row * bias_row * bias_row)
        p_col = p_row.reshape(MAT, 1)
        core = jnp.sum(p_col * p_row * w *
                       (n * E * E + 2.0 * S1 * E + S2)) / (n * n * n)
        ex = renorm(emb32)
        cons = coef * jnp.sum(p_col * (ex - cx) ** 2) / (n * DIM)
        return core + cons

    binmat = (inter_ref[...] != 0.0).astype(jnp.float32)
    pu = presence(uid_ref[...])
    pi = presence(pid_ref[...])
    l1 = glove(pu, ue32_ref[...], cu32_ref[...], cub_ref[...], binmat, ALPHA)
    l2 = glove(pi, ie32_ref[...], ci32_ref[...], cib_ref[...], binmat.T, BETA)
    out_ref[...] = jnp.broadcast_to(l1 + l2, (1, 1))


@jax.jit
def _tc_loss12(uid_col, pid_col, ue32, ie32, cu32, ci32, cub_row, cib_row,
               interactions):
    return pl.pallas_call(
        _tc_loss12_body,
        out_shape=jax.ShapeDtypeStruct((1, 1), jnp.float32),
    )(uid_col, pid_col, ue32, ie32, cu32, ci32, cub_row, cib_row, interactions)


def kernel(user_ids, pos_ids, neg_ids, user_emb, item_emb, co_user_emb,
           co_item_emb, co_user_bias, co_item_bias, interactions):
    loss0_parts = _sc_loss0(user_ids, pos_ids, neg_ids, user_emb, item_emb)
    loss12 = _tc_loss12(
        user_ids.reshape(B, 1), pos_ids.reshape(B, 1),
        user_emb[:MAT], item_emb[:MAT], co_user_emb[:MAT], co_item_emb[:MAT],
        co_user_bias[:MAT].reshape(1, MAT), co_item_bias[:MAT].reshape(1, MAT),
        interactions)
    return jnp.sum(loss0_parts) / (16.0 * B) + loss12[0, 0]
